# baseline (speedup 1.0000x reference)
import functools

import jax
import jax.numpy as jnp
from jax import lax
from jax.experimental import pallas as pl
from jax.experimental.pallas import tpu as pltpu
from jax.experimental.pallas import tpu_sc as plsc

N = 10000
E = 320000
SPATIAL = 12
HIDDEN = 64

NC = 2
NS = 16
NW = NC * NS
EPW = E // NW
CH = 80
NCHUNK = EPW // CH
ZROWS = E // 2

EB = 3200
_EPS = 1e-5


def _node_body(xs_ref, wn_ref, bn_ref, gn_ref, bln_ref, we1_ref, we2_ref,
               be_ref, enc_ref, g_ref, h_ref):
    xm = jnp.dot(xs_ref[...], wn_ref[...],
                 preferred_element_type=jnp.float32) + bn_ref[...]
    m = xm.mean(axis=-1, keepdims=True)
    v = ((xm - m) ** 2).mean(axis=-1, keepdims=True)
    enc = jnp.tanh((xm - m) * lax.rsqrt(v + _EPS) * gn_ref[...] + bln_ref[...])
    enc_ref[...] = enc
    g_ref[...] = jnp.dot(enc, we1_ref[...],
                         preferred_element_type=jnp.float32) + be_ref[...]
    h_ref[...] = jnp.dot(enc, we2_ref[...],
                         preferred_element_type=jnp.float32)


def _node_call(xs, wn, bn, gn, bln, we1, we2, be):
    out_shape = [
        jax.ShapeDtypeStruct((N, HIDDEN), jnp.float32),
        jax.ShapeDtypeStruct((N, HIDDEN), jnp.float32),
        jax.ShapeDtypeStruct((N, HIDDEN), jnp.float32),
    ]
    return pl.pallas_call(_node_body, out_shape=out_shape)(
        xs, wn, bn, gn, bln, we1, we2, be)


def _edge_ln_body(z_ref, ge_ref, be_ref, out_ref):
    z = z_ref[...]
    for p in range(2):
        t = z[:, p * HIDDEN:(p + 1) * HIDDEN]
        m = t.mean(axis=-1, keepdims=True)
        v = ((t - m) ** 2).mean(axis=-1, keepdims=True)
        out_ref[p] = jnp.tanh(
            (t - m) * lax.rsqrt(v + _EPS) * ge_ref[...] + be_ref[...])


def _edge_ln_call(z128, ge, be):
    return pl.pallas_call(
        _edge_ln_body,
        grid=(ZROWS // EB,),
        in_specs=[
            pl.BlockSpec((EB, 2 * HIDDEN), lambda i: (i, 0)),
            pl.BlockSpec((1, HIDDEN), lambda i: (0, 0)),
            pl.BlockSpec((1, HIDDEN), lambda i: (0, 0)),
        ],
        out_specs=pl.BlockSpec((2, EB, HIDDEN), lambda i: (0, i, 0)),
        out_shape=jax.ShapeDtypeStruct((2, ZROWS, HIDDEN), jnp.float32),
    )(z128, ge, be)


def _gather_body(g_hbm, h_hbm, s_hbm, e_hbm, out_hbm, sidx, eidx, buf, sem):
    wid = lax.axis_index("s") * NC + lax.axis_index("c")
    pltpu.sync_copy(s_hbm.at[wid], sidx)
    pltpu.sync_copy(e_hbm.at[wid], eidx)
    lane = (wid // NS) * HIDDEN
    rowbase0 = (wid % NS) * EPW

    def body(c, carry):
        rowbase = rowbase0 + c * CH
        pltpu.async_copy(g_hbm.at[sidx.at[c]], buf, sem).wait()
        pltpu.async_copy(h_hbm.at[eidx.at[c]], buf, sem, add=True).wait()
        pltpu.sync_copy(
            buf, out_hbm.at[pl.ds(rowbase, CH), pl.ds(lane, HIDDEN)])
        return carry

    lax.fori_loop(0, NCHUNK, body, 0)


_gather_call = functools.partial(
    pl.kernel,
    out_type=jax.ShapeDtypeStruct((ZROWS, 2 * HIDDEN), jnp.float32),
    mesh=plsc.VectorSubcoreMesh(core_axis_name="c", subcore_axis_name="s"),
    compiler_params=pltpu.CompilerParams(use_tc_tiling_on_sc=False),
    scratch_types=[
        pltpu.VMEM((NCHUNK, CH), jnp.int32),
        pltpu.VMEM((NCHUNK, CH), jnp.int32),
        pltpu.VMEM((CH, HIDDEN), jnp.float32),
        pltpu.SemaphoreType.DMA,
    ],
)(_gather_body)


def kernel(x, edge_index, Wn, bn, ln_g_n, ln_b_n, We, be, ln_g_e, ln_b_e):
    xs = x[:, :SPATIAL]
    enc, g_tab, h_tab = _node_call(
        xs, Wn, bn.reshape(1, -1), ln_g_n.reshape(1, -1),
        ln_b_n.reshape(1, -1), We[:HIDDEN], We[HIDDEN:], be.reshape(1, -1))
    s3 = edge_index[0].reshape(NW, NCHUNK, CH)
    e3 = edge_index[1].reshape(NW, NCHUNK, CH)
    z128 = _gather_call(g_tab, h_tab, s3, e3)
    out3 = _edge_ln_call(z128, ln_g_e.reshape(1, -1), ln_b_e.reshape(1, -1))
    return (enc, out3.reshape(E, HIDDEN))
